# baseline (speedup 1.0000x reference)
import functools
import jax
import jax.numpy as jnp
from jax import lax
from jax.experimental import pallas as pl
from jax.experimental.pallas import tpu as pltpu
from jax.experimental.pallas import tpu_sc as plsc

N = 10000
E = 320000
R = 16
D = 128
HID = 128
NUM_CLASSES = 40
EPS = 1e-5

NC = 2
NS = 16
NW = NC * NS
EPT = E // NW
CH = 80
NCHUNK = EPT // CH
NR = N * R
ZR = 80
NZB = N // ZR
NPAIR = NCHUNK // 2

_mesh = functools.partial(
    plsc.VectorSubcoreMesh, core_axis_name="c", subcore_axis_name="s")


def _counts_body(dst_hbm, et_hbm, zc_hbm, out_hbm,
                 dbig, ebig, ones, zv, sem, cnt_sh):
  cid = lax.axis_index("c")
  sid = lax.axis_index("s")
  wid = cid * NS + sid

  pltpu.sync_copy(zc_hbm, zv)
  pltpu.sync_copy(zv, cnt_sh.at[pl.ds(sid * (NR // NS), NR // NS)])
  pltpu.sync_copy(dst_hbm.at[wid, pl.ds(0, NCHUNK)], dbig)
  pltpu.sync_copy(et_hbm.at[wid, pl.ds(0, NCHUNK)], ebig)

  for t in range(CH // 16):
    ones[0, pl.ds(t * 16, 16)] = jnp.full((16,), 1.0, jnp.float32)

  def prep(g, _):
    for t in range(CH // 16):
      sl = pl.ds(t * 16, 16)
      ebig[g, sl] = dbig[g, sl] * R + ebig[g, sl]
    return _

  lax.fori_loop(0, NCHUNK, prep, None)
  plsc.subcore_barrier()

  def fire(g, _):
    pltpu.async_copy(ones.at[0], cnt_sh.at[ebig.at[g]], sem, add=True)

    @pl.when(g >= 8)
    def _():
      pltpu.make_async_copy(ones.at[0], cnt_sh.at[ebig.at[g - 8]],
                            sem).wait()

    return _

  lax.fori_loop(0, NCHUNK, fire, None)

  def drain(g, _):
    pltpu.make_async_copy(ones.at[0], cnt_sh.at[ebig.at[g]], sem).wait()
    return _

  lax.fori_loop(NCHUNK - 8, NCHUNK, drain, None)
  plsc.subcore_barrier()
  pltpu.sync_copy(cnt_sh.at[pl.ds(sid * (NR // NS), NR // NS)], zv)
  pltpu.sync_copy(zv,
                  out_hbm.at[pl.ds(cid * NR + sid * (NR // NS), NR // NS)])


def _counts(dst3, et3, zc):
  k = pl.kernel(
      _counts_body,
      out_type=jax.ShapeDtypeStruct((NC * NR,), jnp.float32),
      mesh=_mesh(),
      scratch_types=[
          pltpu.VMEM((NCHUNK, CH), jnp.int32),
          pltpu.VMEM((NCHUNK, CH), jnp.int32),
          pltpu.VMEM((1, CH), jnp.float32),
          pltpu.VMEM((NR // NS,), jnp.float32),
          pltpu.SemaphoreType.DMA,
          pltpu.VMEM_SHARED((NR,), jnp.float32),
      ],
  )
  return k(dst3, et3, zc)


def _agg_body(y_hbm, meta_hbm, invc_hbm, za_hbm, out_hbm,
              mb0, mb1, mb2, mb3, rix0, rix1, rix2, rix3,
              wix0, wix1, wix2, wix3, div0, div1, div2, div3,
              wv0, wv1, wv2, wv3, rg0, rg1, rg2, rg3,
              m0, m1, m2, m3, g0, g1, g2, g3,
              s0, s1, s2, s3, w0, w1, w2, w3, acc_sh):
  cid = lax.axis_index("c")
  sid = lax.axis_index("s")
  wid = cid * NS + sid

  mbs = (mb0, mb1, mb2, mb3)
  msems = (m0, m1, m2, m3)
  rixs = (rix0, rix1, rix2, rix3)
  wixs = (wix0, wix1, wix2, wix3)
  divs = (div0, div1, div2, div3)
  wvs = (wv0, wv1, wv2, wv3)
  rgs = (rg0, rg1, rg2, rg3)
  gsems = (g0, g1, g2, g3)
  ssems = (s0, s1, s2, s3)
  wsems = (w0, w1, w2, w3)

  pltpu.sync_copy(za_hbm, rg0)
  for z in range(pl.cdiv(NZB, NS)):
    blk = sid + z * NS

    @pl.when(blk < NZB)
    def _():
      pltpu.sync_copy(rg0, acc_sh.at[pl.ds(blk * ZR, ZR)])

  def meta_issue(c, j):
    pltpu.async_copy(meta_hbm.at[wid, c], mbs[j], msems[j])

  def prep_issue(c, j):
    pltpu.make_async_copy(meta_hbm.at[wid, c], mbs[j], msems[j]).wait()
    for t in range(CH // 16):
      sl = pl.ds(t * 16, 16)
      rixs[j][sl] = mbs[j][2, sl] * N + mbs[j][0, sl]
      wixs[j][sl] = mbs[j][1, sl] * R + mbs[j][2, sl]
    pltpu.async_copy(invc_hbm.at[wixs[j]], wvs[j], wsems[j])
    pltpu.async_copy(y_hbm.at[rixs[j]], rgs[j], gsems[j])

  def process(c, q, first=False, last=False):
    qn = (q + 2) % 4
    pltpu.make_async_copy(y_hbm.at[rixs[q]], rgs[q], gsems[q]).wait()
    pltpu.make_async_copy(invc_hbm.at[wixs[q]], wvs[q], wsems[q]).wait()
    if not first:
      pltpu.make_async_copy(rgs[qn], acc_sh.at[divs[qn]], ssems[qn]).wait()
    if not last:
      @pl.when(c + 2 < NCHUNK)
      def _():
        prep_issue(c + 2, qn)

    for t in range(CH // 16):
      sl = pl.ds(t * 16, 16)
      divs[q][sl] = mbs[q][1, sl]
    if not last:
      @pl.when(c + 4 < NCHUNK)
      def _():
        meta_issue(c + 4, q)

    for t in range(CH // 16):
      sl = pl.ds(t * 16, 16)
      w16 = wvs[q][sl]
      for ii in range(16):
        w = w16[ii]
        for jj in range(D // 16):
          fl = pl.ds(jj * 16, 16)
          rgs[q][t * 16 + ii, fl] = rgs[q][t * 16 + ii, fl] * w
    pltpu.async_copy(rgs[q], acc_sh.at[divs[q]], ssems[q], add=True)

  for j in range(4):
    meta_issue(j, j)
  prep_issue(0, 0)
  prep_issue(1, 1)
  plsc.subcore_barrier()
  process(0, 0, first=True)
  process(1, 1, first=True)
  process(2, 2)
  process(3, 3)

  def quad(k, _):
    c = k * 4
    process(c, 0)
    process(c + 1, 1)
    process(c + 2, 2)
    process(c + 3, 3)
    return _

  lax.fori_loop(1, NCHUNK // 4, quad, None)
  process(NCHUNK - 1, 0, last=True)
  pltpu.make_async_copy(rgs[3], acc_sh.at[divs[3]], ssems[3]).wait()
  pltpu.make_async_copy(rgs[0], acc_sh.at[divs[0]], ssems[0]).wait()
  plsc.subcore_barrier()
  for z in range(pl.cdiv(NZB, NS)):
    blk = sid + z * NS

    @pl.when(blk < NZB)
    def _():
      pltpu.sync_copy(acc_sh.at[pl.ds(blk * ZR, ZR)], rg0)
      pltpu.sync_copy(rg0, out_hbm.at[cid, pl.ds(blk * ZR, ZR)])


def _aggregate(y, meta, invc, za):
  k = pl.kernel(
      _agg_body,
      out_type=jax.ShapeDtypeStruct((NC, N, D), jnp.float32),
      mesh=_mesh(),
      scratch_types=[
          pltpu.VMEM((3, CH), jnp.int32),
          pltpu.VMEM((3, CH), jnp.int32),
          pltpu.VMEM((3, CH), jnp.int32),
          pltpu.VMEM((3, CH), jnp.int32),
          pltpu.VMEM((CH,), jnp.int32),
          pltpu.VMEM((CH,), jnp.int32),
          pltpu.VMEM((CH,), jnp.int32),
          pltpu.VMEM((CH,), jnp.int32),
          pltpu.VMEM((CH,), jnp.int32),
          pltpu.VMEM((CH,), jnp.int32),
          pltpu.VMEM((CH,), jnp.int32),
          pltpu.VMEM((CH,), jnp.int32),
          pltpu.VMEM((CH,), jnp.int32),
          pltpu.VMEM((CH,), jnp.int32),
          pltpu.VMEM((CH,), jnp.int32),
          pltpu.VMEM((CH,), jnp.int32),
          pltpu.VMEM((CH,), jnp.float32),
          pltpu.VMEM((CH,), jnp.float32),
          pltpu.VMEM((CH,), jnp.float32),
          pltpu.VMEM((CH,), jnp.float32),
          pltpu.VMEM((CH, D), jnp.float32),
          pltpu.VMEM((CH, D), jnp.float32),
          pltpu.VMEM((CH, D), jnp.float32),
          pltpu.VMEM((CH, D), jnp.float32),
          pltpu.SemaphoreType.DMA,
          pltpu.SemaphoreType.DMA,
          pltpu.SemaphoreType.DMA,
          pltpu.SemaphoreType.DMA,
          pltpu.SemaphoreType.DMA,
          pltpu.SemaphoreType.DMA,
          pltpu.SemaphoreType.DMA,
          pltpu.SemaphoreType.DMA,
          pltpu.SemaphoreType.DMA,
          pltpu.SemaphoreType.DMA,
          pltpu.SemaphoreType.DMA,
          pltpu.SemaphoreType.DMA,
          pltpu.SemaphoreType.DMA,
          pltpu.SemaphoreType.DMA,
          pltpu.SemaphoreType.DMA,
          pltpu.SemaphoreType.DMA,
          pltpu.VMEM_SHARED((N, D), jnp.float32),
      ],
  )
  return k(y, meta, invc, za)


NB = 10
BN_ROWS = N // NB


def _inv_body(c_ref, o_ref):
  c = c_ref[0] + c_ref[1]
  o_ref[...] = 1.0 / jnp.maximum(c, 1.0)


def _inv_counts(counts):
  c2 = counts.reshape(NC, NR // 128, 128)
  out = pl.pallas_call(
      _inv_body,
      out_shape=jax.ShapeDtypeStruct((NR // 128, 128), jnp.float32),
  )(c2)
  return out.reshape(NR)


def _relmm_body(x_ref, w_ref, y_ref):
  y_ref[...] = jnp.dot(x_ref[...], w_ref[0],
                       preferred_element_type=jnp.float32)


def _rel_matmul(x, wrel):
  dout = wrel.shape[2]
  return pl.pallas_call(
      _relmm_body,
      grid=(R,),
      in_specs=[
          pl.BlockSpec((N, D), lambda r: (0, 0)),
          pl.BlockSpec((1, D, dout), lambda r: (r, 0, 0)),
      ],
      out_specs=pl.BlockSpec((N, dout), lambda r: (r, 0)),
      out_shape=jax.ShapeDtypeStruct((R * N, dout), jnp.float32),
  )(x, wrel)


def _pre_body(a_ref, x_ref, w_ref, b_ref, h_ref, p_ref):
  h = a_ref[0] + a_ref[1] + jnp.dot(x_ref[...], w_ref[...],
                                    preferred_element_type=jnp.float32)
  h = h + b_ref[...]
  h_ref[...] = h
  s = jnp.sum(h, axis=0, keepdims=True)
  ss = jnp.sum(h * h, axis=0, keepdims=True)
  p_ref[...] = jnp.concatenate(
      [s, ss, jnp.zeros((6, HID), jnp.float32)], axis=0)[None]


def _pre_bn(a, x, wroot, b):
  return pl.pallas_call(
      _pre_body,
      grid=(NB,),
      in_specs=[
          pl.BlockSpec((NC, BN_ROWS, HID), lambda i: (0, i, 0)),
          pl.BlockSpec((BN_ROWS, D), lambda i: (i, 0)),
          pl.BlockSpec((D, HID), lambda i: (0, 0)),
          pl.BlockSpec((1, HID), lambda i: (0, 0)),
      ],
      out_specs=[
          pl.BlockSpec((BN_ROWS, HID), lambda i: (i, 0)),
          pl.BlockSpec((1, 8, HID), lambda i: (i, 0, 0)),
      ],
      out_shape=[
          jax.ShapeDtypeStruct((N, HID), jnp.float32),
          jax.ShapeDtypeStruct((NB, 8, HID), jnp.float32),
      ],
  )(a, x, wroot, b)


def _bn_relu_body(h_ref, p_ref, g_ref, be_ref, o_ref):
  s = jnp.sum(p_ref[:, 0, :], axis=0)
  ss = jnp.sum(p_ref[:, 1, :], axis=0)
  m = s / N
  v = ss / N - m * m
  y = (h_ref[...] - m) * lax.rsqrt(v + EPS) * g_ref[...] + be_ref[...]
  o_ref[...] = jnp.maximum(y, 0.0)


def _bn_relu(h, parts, g, be):
  return pl.pallas_call(
      _bn_relu_body,
      grid=(NB,),
      in_specs=[
          pl.BlockSpec((BN_ROWS, HID), lambda i: (i, 0)),
          pl.BlockSpec((NB, 8, HID), lambda i: (0, 0, 0)),
          pl.BlockSpec((1, HID), lambda i: (0, 0)),
          pl.BlockSpec((1, HID), lambda i: (0, 0)),
      ],
      out_specs=pl.BlockSpec((BN_ROWS, HID), lambda i: (i, 0)),
      out_shape=jax.ShapeDtypeStruct((N, HID), jnp.float32),
  )(h, parts, g, be)


def _bn_relu_fc_body(h_ref, p_ref, g_ref, be_ref, wf_ref, bf_ref, o_ref):
  s = jnp.sum(p_ref[:, 0, :], axis=0)
  ss = jnp.sum(p_ref[:, 1, :], axis=0)
  m = s / N
  v = ss / N - m * m
  y = (h_ref[...] - m) * lax.rsqrt(v + EPS) * g_ref[...] + be_ref[...]
  y = jnp.maximum(y, 0.0)
  o_ref[...] = jnp.dot(y, wf_ref[...],
                       preferred_element_type=jnp.float32) + bf_ref[...]


def _bn_relu_fc(h, parts, g, be, wf, bf):
  return pl.pallas_call(
      _bn_relu_fc_body,
      grid=(NB,),
      in_specs=[
          pl.BlockSpec((BN_ROWS, HID), lambda i: (i, 0)),
          pl.BlockSpec((NB, 8, HID), lambda i: (0, 0, 0)),
          pl.BlockSpec((1, HID), lambda i: (0, 0)),
          pl.BlockSpec((1, HID), lambda i: (0, 0)),
          pl.BlockSpec((HID, NUM_CLASSES), lambda i: (0, 0)),
          pl.BlockSpec((1, NUM_CLASSES), lambda i: (0, 0)),
      ],
      out_specs=pl.BlockSpec((BN_ROWS, NUM_CLASSES), lambda i: (i, 0)),
      out_shape=jax.ShapeDtypeStruct((N, NUM_CLASSES), jnp.float32),
  )(h, parts, g, be, wf, bf)


def kernel(x, edge_index, edge_attr, Wrel1, Wroot1, b1, g1, be1,
           Wrel2, Wroot2, b2, g2, be2, Wf, bf):
  src = edge_index[0].astype(jnp.int32).reshape(NW, NCHUNK, CH)
  dst = edge_index[1].astype(jnp.int32).reshape(NW, NCHUNK, CH)
  et = edge_attr.astype(jnp.int32).reshape(NW, NCHUNK, CH)
  meta = jnp.stack([src, dst, et], axis=2)

  zc = jnp.zeros((NR // NS,), jnp.float32)
  za = jnp.zeros((ZR, D), jnp.float32)

  counts = _counts(dst, et, zc)
  invc = _inv_counts(counts)

  b1r = b1.reshape(1, HID)
  g1r = g1.reshape(1, HID)
  be1r = be1.reshape(1, HID)
  b2r = b2.reshape(1, HID)
  g2r = g2.reshape(1, HID)
  be2r = be2.reshape(1, HID)
  bfr = bf.reshape(1, NUM_CLASSES)

  y1 = _rel_matmul(x, Wrel1)
  a1 = _aggregate(y1, meta, invc, za)
  h1p, p1 = _pre_bn(a1, x, Wroot1, b1r)
  h1 = _bn_relu(h1p, p1, g1r, be1r)

  y2 = _rel_matmul(h1, Wrel2)
  a2 = _aggregate(y2, meta, invc, za)
  h2p, p2 = _pre_bn(a2, h1, Wroot2, b2r)
  out = _bn_relu_fc(h2p, p2, g2r, be2r, Wf, bfr)
  return out


# speedup vs baseline: 1.0371x; 1.0371x over previous
import functools
import jax
import jax.numpy as jnp
from jax import lax
from jax.experimental import pallas as pl
from jax.experimental.pallas import tpu as pltpu
from jax.experimental.pallas import tpu_sc as plsc

N = 10000
E = 320000
R = 16
D = 128
HID = 128
NUM_CLASSES = 40
EPS = 1e-5

NC = 2
NS = 16
NW = NC * NS
EPT = E // NW
CH = 80
NCHUNK = EPT // CH
NR = N * R
ZR = 80
NZB = N // ZR
NPAIR = NCHUNK // 2

_mesh = functools.partial(
    plsc.VectorSubcoreMesh, core_axis_name="c", subcore_axis_name="s")


def _counts_body(src_hbm, dst_hbm, et_hbm, zc_hbm, out_hbm, meta_hbm,
                 sbig, dbig, ebig, wxbig, meta_v, ones, zv, sem, cnt_sh):
  cid = lax.axis_index("c")
  sid = lax.axis_index("s")
  wid = cid * NS + sid

  pltpu.sync_copy(zc_hbm, zv)
  pltpu.sync_copy(zv, cnt_sh.at[pl.ds(sid * (NR // NS), NR // NS)])
  pltpu.sync_copy(src_hbm.at[pl.ds(wid * EPT, EPT)], sbig)
  pltpu.sync_copy(dst_hbm.at[pl.ds(wid * EPT, EPT)], dbig)
  pltpu.sync_copy(et_hbm.at[pl.ds(wid * EPT, EPT)], ebig)

  for t in range(CH // 16):
    ones[0, pl.ds(t * 16, 16)] = jnp.full((16,), 1.0, jnp.float32)

  def prep(g, _):
    for t in range(CH // 16):
      sl = pl.ds(g * CH + t * 16, 16)
      sv = sbig[sl]
      dv = dbig[sl]
      ev = ebig[sl]
      meta_v[pl.ds(g * 3 * CH + t * 16, 16)] = sv
      meta_v[pl.ds(g * 3 * CH + CH + t * 16, 16)] = dv
      meta_v[pl.ds(g * 3 * CH + 2 * CH + t * 16, 16)] = ev
      wxbig[g, pl.ds(t * 16, 16)] = dv * R + ev
    return _

  lax.fori_loop(0, NCHUNK, prep, None)
  pltpu.sync_copy(meta_v, meta_hbm.at[pl.ds(wid * NCHUNK * 3 * CH,
                                            NCHUNK * 3 * CH)])
  plsc.subcore_barrier()

  def fire(g, _):
    pltpu.async_copy(ones.at[0], cnt_sh.at[wxbig.at[g]], sem, add=True)

    @pl.when(g >= 8)
    def _():
      pltpu.make_async_copy(ones.at[0], cnt_sh.at[wxbig.at[g - 8]],
                            sem).wait()

    return _

  lax.fori_loop(0, NCHUNK, fire, None)

  def drain(g, _):
    pltpu.make_async_copy(ones.at[0], cnt_sh.at[wxbig.at[g]], sem).wait()
    return _

  lax.fori_loop(NCHUNK - 8, NCHUNK, drain, None)
  plsc.subcore_barrier()
  pltpu.sync_copy(cnt_sh.at[pl.ds(sid * (NR // NS), NR // NS)], zv)
  pltpu.sync_copy(zv,
                  out_hbm.at[pl.ds(cid * NR + sid * (NR // NS), NR // NS)])


def _counts(src1, dst1, et1, zc):
  k = pl.kernel(
      _counts_body,
      name="counts_sc",
      out_type=[
          jax.ShapeDtypeStruct((NC * NR,), jnp.float32),
          jax.ShapeDtypeStruct((NW * NCHUNK * 3 * CH,), jnp.int32),
      ],
      mesh=_mesh(),
      scratch_types=[
          pltpu.VMEM((EPT,), jnp.int32),
          pltpu.VMEM((EPT,), jnp.int32),
          pltpu.VMEM((EPT,), jnp.int32),
          pltpu.VMEM((NCHUNK, CH), jnp.int32),
          pltpu.VMEM((NCHUNK * 3 * CH,), jnp.int32),
          pltpu.VMEM((1, CH), jnp.float32),
          pltpu.VMEM((NR // NS,), jnp.float32),
          pltpu.SemaphoreType.DMA,
          pltpu.VMEM_SHARED((NR,), jnp.float32),
      ],
  )
  return k(src1, dst1, et1, zc)


def _agg_body(y_hbm, meta_hbm, invc_hbm, za_hbm, out_hbm,
              mb0, mb1, mb2, mb3, rix0, rix1, rix2, rix3,
              wix0, wix1, wix2, wix3, div0, div1, div2, div3,
              wv0, wv1, wv2, wv3, rg0, rg1, rg2, rg3,
              m0, m1, m2, m3, g0, g1, g2, g3,
              s0, s1, s2, s3, w0, w1, w2, w3, acc_sh):
  cid = lax.axis_index("c")
  sid = lax.axis_index("s")
  wid = cid * NS + sid

  mbs = (mb0, mb1, mb2, mb3)
  msems = (m0, m1, m2, m3)
  rixs = (rix0, rix1, rix2, rix3)
  wixs = (wix0, wix1, wix2, wix3)
  divs = (div0, div1, div2, div3)
  wvs = (wv0, wv1, wv2, wv3)
  rgs = (rg0, rg1, rg2, rg3)
  gsems = (g0, g1, g2, g3)
  ssems = (s0, s1, s2, s3)
  wsems = (w0, w1, w2, w3)

  pltpu.sync_copy(za_hbm, rg0)
  for z in range(pl.cdiv(NZB, NS)):
    blk = sid + z * NS

    @pl.when(blk < NZB)
    def _():
      pltpu.sync_copy(rg0, acc_sh.at[pl.ds(blk * ZR, ZR)])

  def meta_issue(c, j):
    pltpu.async_copy(meta_hbm.at[wid, c], mbs[j], msems[j])

  def prep_issue(c, j):
    pltpu.make_async_copy(meta_hbm.at[wid, c], mbs[j], msems[j]).wait()
    for t in range(CH // 16):
      sl = pl.ds(t * 16, 16)
      rixs[j][sl] = mbs[j][2, sl] * N + mbs[j][0, sl]
      wixs[j][sl] = mbs[j][1, sl] * R + mbs[j][2, sl]
    pltpu.async_copy(invc_hbm.at[wixs[j]], wvs[j], wsems[j])
    pltpu.async_copy(y_hbm.at[rixs[j]], rgs[j], gsems[j])

  def process(c, q, first=False, last=False):
    qn = (q + 2) % 4
    pltpu.make_async_copy(y_hbm.at[rixs[q]], rgs[q], gsems[q]).wait()
    pltpu.make_async_copy(invc_hbm.at[wixs[q]], wvs[q], wsems[q]).wait()
    if not first:
      pltpu.make_async_copy(rgs[qn], acc_sh.at[divs[qn]], ssems[qn]).wait()
    if not last:
      @pl.when(c + 2 < NCHUNK)
      def _():
        prep_issue(c + 2, qn)

    for t in range(CH // 16):
      sl = pl.ds(t * 16, 16)
      divs[q][sl] = mbs[q][1, sl]
    if not last:
      @pl.when(c + 4 < NCHUNK)
      def _():
        meta_issue(c + 4, q)

    for t in range(CH // 16):
      sl = pl.ds(t * 16, 16)
      w16 = wvs[q][sl]
      for ii in range(16):
        w = w16[ii]
        for jj in range(D // 16):
          fl = pl.ds(jj * 16, 16)
          rgs[q][t * 16 + ii, fl] = rgs[q][t * 16 + ii, fl] * w
    pltpu.async_copy(rgs[q], acc_sh.at[divs[q]], ssems[q], add=True)

  for j in range(4):
    meta_issue(j, j)
  prep_issue(0, 0)
  prep_issue(1, 1)
  plsc.subcore_barrier()
  process(0, 0, first=True)
  process(1, 1, first=True)
  process(2, 2)
  process(3, 3)

  def quad(k, _):
    c = k * 4
    process(c, 0)
    process(c + 1, 1)
    process(c + 2, 2)
    process(c + 3, 3)
    return _

  lax.fori_loop(1, NCHUNK // 4, quad, None)
  process(NCHUNK - 1, 0, last=True)
  pltpu.make_async_copy(rgs[3], acc_sh.at[divs[3]], ssems[3]).wait()
  pltpu.make_async_copy(rgs[0], acc_sh.at[divs[0]], ssems[0]).wait()
  plsc.subcore_barrier()
  for z in range(pl.cdiv(NZB, NS)):
    blk = sid + z * NS

    @pl.when(blk < NZB)
    def _():
      pltpu.sync_copy(acc_sh.at[pl.ds(blk * ZR, ZR)], rg0)
      pltpu.sync_copy(rg0, out_hbm.at[cid, pl.ds(blk * ZR, ZR)])


def _aggregate(y, meta, invc, za):
  k = pl.kernel(
      _agg_body,
      name="agg_sc",
      out_type=jax.ShapeDtypeStruct((NC, N, D), jnp.float32),
      mesh=_mesh(),
      scratch_types=[
          pltpu.VMEM((3, CH), jnp.int32),
          pltpu.VMEM((3, CH), jnp.int32),
          pltpu.VMEM((3, CH), jnp.int32),
          pltpu.VMEM((3, CH), jnp.int32),
          pltpu.VMEM((CH,), jnp.int32),
          pltpu.VMEM((CH,), jnp.int32),
          pltpu.VMEM((CH,), jnp.int32),
          pltpu.VMEM((CH,), jnp.int32),
          pltpu.VMEM((CH,), jnp.int32),
          pltpu.VMEM((CH,), jnp.int32),
          pltpu.VMEM((CH,), jnp.int32),
          pltpu.VMEM((CH,), jnp.int32),
          pltpu.VMEM((CH,), jnp.int32),
          pltpu.VMEM((CH,), jnp.int32),
          pltpu.VMEM((CH,), jnp.int32),
          pltpu.VMEM((CH,), jnp.int32),
          pltpu.VMEM((CH,), jnp.float32),
          pltpu.VMEM((CH,), jnp.float32),
          pltpu.VMEM((CH,), jnp.float32),
          pltpu.VMEM((CH,), jnp.float32),
          pltpu.VMEM((CH, D), jnp.float32),
          pltpu.VMEM((CH, D), jnp.float32),
          pltpu.VMEM((CH, D), jnp.float32),
          pltpu.VMEM((CH, D), jnp.float32),
          pltpu.SemaphoreType.DMA,
          pltpu.SemaphoreType.DMA,
          pltpu.SemaphoreType.DMA,
          pltpu.SemaphoreType.DMA,
          pltpu.SemaphoreType.DMA,
          pltpu.SemaphoreType.DMA,
          pltpu.SemaphoreType.DMA,
          pltpu.SemaphoreType.DMA,
          pltpu.SemaphoreType.DMA,
          pltpu.SemaphoreType.DMA,
          pltpu.SemaphoreType.DMA,
          pltpu.SemaphoreType.DMA,
          pltpu.SemaphoreType.DMA,
          pltpu.SemaphoreType.DMA,
          pltpu.SemaphoreType.DMA,
          pltpu.SemaphoreType.DMA,
          pltpu.VMEM_SHARED((N, D), jnp.float32),
      ],
  )
  return k(y, meta, invc, za)


NB = 10
BN_ROWS = N // NB


def _inv_body(c_ref, o_ref):
  c = c_ref[0] + c_ref[1]
  o_ref[...] = 1.0 / jnp.maximum(c, 1.0)


def _inv_counts(counts):
  c2 = counts.reshape(NC, NR // 128, 128)
  out = pl.pallas_call(
      _inv_body,
      out_shape=jax.ShapeDtypeStruct((NR // 128, 128), jnp.float32),
  )(c2)
  return out.reshape(NR)


def _relmm_body(x_ref, w_ref, y_ref):
  y_ref[...] = jnp.dot(x_ref[...], w_ref[0],
                       preferred_element_type=jnp.float32)


def _rel_matmul(x, wrel):
  dout = wrel.shape[2]
  return pl.pallas_call(
      _relmm_body,
      grid=(R,),
      in_specs=[
          pl.BlockSpec((N, D), lambda r: (0, 0)),
          pl.BlockSpec((1, D, dout), lambda r: (r, 0, 0)),
      ],
      out_specs=pl.BlockSpec((N, dout), lambda r: (r, 0)),
      out_shape=jax.ShapeDtypeStruct((R * N, dout), jnp.float32),
  )(x, wrel)


def _pre_body(a_ref, x_ref, w_ref, b_ref, h_ref, p_ref):
  h = a_ref[0] + a_ref[1] + jnp.dot(x_ref[...], w_ref[...],
                                    preferred_element_type=jnp.float32)
  h = h + b_ref[...]
  h_ref[...] = h
  s = jnp.sum(h, axis=0, keepdims=True)
  ss = jnp.sum(h * h, axis=0, keepdims=True)
  p_ref[...] = jnp.concatenate(
      [s, ss, jnp.zeros((6, HID), jnp.float32)], axis=0)[None]


def _pre_bn(a, x, wroot, b):
  return pl.pallas_call(
      _pre_body,
      grid=(NB,),
      in_specs=[
          pl.BlockSpec((NC, BN_ROWS, HID), lambda i: (0, i, 0)),
          pl.BlockSpec((BN_ROWS, D), lambda i: (i, 0)),
          pl.BlockSpec((D, HID), lambda i: (0, 0)),
          pl.BlockSpec((1, HID), lambda i: (0, 0)),
      ],
      out_specs=[
          pl.BlockSpec((BN_ROWS, HID), lambda i: (i, 0)),
          pl.BlockSpec((1, 8, HID), lambda i: (i, 0, 0)),
      ],
      out_shape=[
          jax.ShapeDtypeStruct((N, HID), jnp.float32),
          jax.ShapeDtypeStruct((NB, 8, HID), jnp.float32),
      ],
  )(a, x, wroot, b)


def _bn_relu_body(h_ref, p_ref, g_ref, be_ref, o_ref):
  s = jnp.sum(p_ref[:, 0, :], axis=0)
  ss = jnp.sum(p_ref[:, 1, :], axis=0)
  m = s / N
  v = ss / N - m * m
  y = (h_ref[...] - m) * lax.rsqrt(v + EPS) * g_ref[...] + be_ref[...]
  o_ref[...] = jnp.maximum(y, 0.0)


def _bn_relu(h, parts, g, be):
  return pl.pallas_call(
      _bn_relu_body,
      grid=(NB,),
      in_specs=[
          pl.BlockSpec((BN_ROWS, HID), lambda i: (i, 0)),
          pl.BlockSpec((NB, 8, HID), lambda i: (0, 0, 0)),
          pl.BlockSpec((1, HID), lambda i: (0, 0)),
          pl.BlockSpec((1, HID), lambda i: (0, 0)),
      ],
      out_specs=pl.BlockSpec((BN_ROWS, HID), lambda i: (i, 0)),
      out_shape=jax.ShapeDtypeStruct((N, HID), jnp.float32),
  )(h, parts, g, be)


def _bn_relu_fc_body(h_ref, p_ref, g_ref, be_ref, wf_ref, bf_ref, o_ref):
  s = jnp.sum(p_ref[:, 0, :], axis=0)
  ss = jnp.sum(p_ref[:, 1, :], axis=0)
  m = s / N
  v = ss / N - m * m
  y = (h_ref[...] - m) * lax.rsqrt(v + EPS) * g_ref[...] + be_ref[...]
  y = jnp.maximum(y, 0.0)
  o_ref[...] = jnp.dot(y, wf_ref[...],
                       preferred_element_type=jnp.float32) + bf_ref[...]


def _bn_relu_fc(h, parts, g, be, wf, bf):
  return pl.pallas_call(
      _bn_relu_fc_body,
      grid=(NB,),
      in_specs=[
          pl.BlockSpec((BN_ROWS, HID), lambda i: (i, 0)),
          pl.BlockSpec((NB, 8, HID), lambda i: (0, 0, 0)),
          pl.BlockSpec((1, HID), lambda i: (0, 0)),
          pl.BlockSpec((1, HID), lambda i: (0, 0)),
          pl.BlockSpec((HID, NUM_CLASSES), lambda i: (0, 0)),
          pl.BlockSpec((1, NUM_CLASSES), lambda i: (0, 0)),
      ],
      out_specs=pl.BlockSpec((BN_ROWS, NUM_CLASSES), lambda i: (i, 0)),
      out_shape=jax.ShapeDtypeStruct((N, NUM_CLASSES), jnp.float32),
  )(h, parts, g, be, wf, bf)


def kernel(x, edge_index, edge_attr, Wrel1, Wroot1, b1, g1, be1,
           Wrel2, Wroot2, b2, g2, be2, Wf, bf):
  src = edge_index[0].astype(jnp.int32)
  dst = edge_index[1].astype(jnp.int32)
  et = edge_attr.astype(jnp.int32)
  zc = jnp.zeros((NR // NS,), jnp.float32)
  za = jnp.zeros((ZR, D), jnp.float32)

  counts, meta = _counts(src, dst, et, zc)
  meta = meta.reshape(NW, NCHUNK, 3, CH)
  invc = _inv_counts(counts)

  b1r = b1.reshape(1, HID)
  g1r = g1.reshape(1, HID)
  be1r = be1.reshape(1, HID)
  b2r = b2.reshape(1, HID)
  g2r = g2.reshape(1, HID)
  be2r = be2.reshape(1, HID)
  bfr = bf.reshape(1, NUM_CLASSES)

  y1 = _rel_matmul(x, Wrel1)
  a1 = _aggregate(y1, meta, invc, za)
  h1p, p1 = _pre_bn(a1, x, Wroot1, b1r)
  h1 = _bn_relu(h1p, p1, g1r, be1r)

  y2 = _rel_matmul(h1, Wrel2)
  a2 = _aggregate(y2, meta, invc, za)
  h2p, p2 = _pre_bn(a2, h1, Wroot2, b2r)
  out = _bn_relu_fc(h2p, p2, g2r, be2r, Wf, bfr)
  return out


# speedup vs baseline: 1.0578x; 1.0200x over previous
import functools
import jax
import jax.numpy as jnp
from jax import lax
from jax.experimental import pallas as pl
from jax.experimental.pallas import tpu as pltpu
from jax.experimental.pallas import tpu_sc as plsc

N = 10000
E = 320000
R = 16
D = 128
HID = 128
NUM_CLASSES = 40
EPS = 1e-5

NC = 2
NS = 16
NW = NC * NS
EPT = E // NW
CH = 80
NCHUNK = EPT // CH
NR = N * R
ZR = 80
NZB = N // ZR
NPAIR = NCHUNK // 2

_mesh = functools.partial(
    plsc.VectorSubcoreMesh, core_axis_name="c", subcore_axis_name="s")


def _counts_body(src_hbm, dst_hbm, et_hbm, zc_hbm, out_hbm, meta_hbm,
                 sbig, dbig, ebig, wxbig, meta_v, ones, zv, sem, cnt_sh):
  cid = lax.axis_index("c")
  sid = lax.axis_index("s")
  wid = cid * NS + sid

  pltpu.sync_copy(zc_hbm, zv)
  pltpu.sync_copy(zv, cnt_sh.at[pl.ds(sid * (NR // NS), NR // NS)])
  pltpu.sync_copy(src_hbm.at[pl.ds(wid * EPT, EPT)], sbig)
  pltpu.sync_copy(dst_hbm.at[pl.ds(wid * EPT, EPT)], dbig)
  pltpu.sync_copy(et_hbm.at[pl.ds(wid * EPT, EPT)], ebig)

  for t in range(CH // 16):
    ones[0, pl.ds(t * 16, 16)] = jnp.full((16,), 1.0, jnp.float32)

  def prep(g, _):
    for t in range(CH // 16):
      sl = pl.ds(g * CH + t * 16, 16)
      sv = sbig[sl]
      dv = dbig[sl]
      ev = ebig[sl]
      meta_v[pl.ds(g * 3 * CH + t * 16, 16)] = sv
      meta_v[pl.ds(g * 3 * CH + CH + t * 16, 16)] = dv
      meta_v[pl.ds(g * 3 * CH + 2 * CH + t * 16, 16)] = ev
      wxbig[g, pl.ds(t * 16, 16)] = dv * R + ev
    return _

  lax.fori_loop(0, NCHUNK, prep, None)
  pltpu.sync_copy(meta_v, meta_hbm.at[pl.ds(wid * NCHUNK * 3 * CH,
                                            NCHUNK * 3 * CH)])
  plsc.subcore_barrier()

  def fire(g, _):
    pltpu.async_copy(ones.at[0], cnt_sh.at[wxbig.at[g]], sem, add=True)

    @pl.when(g >= 8)
    def _():
      pltpu.make_async_copy(ones.at[0], cnt_sh.at[wxbig.at[g - 8]],
                            sem).wait()

    return _

  lax.fori_loop(0, NCHUNK, fire, None)

  def drain(g, _):
    pltpu.make_async_copy(ones.at[0], cnt_sh.at[wxbig.at[g]], sem).wait()
    return _

  lax.fori_loop(NCHUNK - 8, NCHUNK, drain, None)
  plsc.subcore_barrier()
  pltpu.sync_copy(cnt_sh.at[pl.ds(sid * (NR // NS), NR // NS)], zv)
  pltpu.sync_copy(zv,
                  out_hbm.at[pl.ds(cid * NR + sid * (NR // NS), NR // NS)])


def _counts(src1, dst1, et1, zc):
  k = pl.kernel(
      _counts_body,
      name="counts_sc",
      out_type=[
          jax.ShapeDtypeStruct((NC * NR,), jnp.float32),
          jax.ShapeDtypeStruct((NW * NCHUNK * 3 * CH,), jnp.int32),
      ],
      mesh=_mesh(),
      scratch_types=[
          pltpu.VMEM((EPT,), jnp.int32),
          pltpu.VMEM((EPT,), jnp.int32),
          pltpu.VMEM((EPT,), jnp.int32),
          pltpu.VMEM((NCHUNK, CH), jnp.int32),
          pltpu.VMEM((NCHUNK * 3 * CH,), jnp.int32),
          pltpu.VMEM((1, CH), jnp.float32),
          pltpu.VMEM((NR // NS,), jnp.float32),
          pltpu.SemaphoreType.DMA,
          pltpu.VMEM_SHARED((NR,), jnp.float32),
      ],
  )
  return k(src1, dst1, et1, zc)


def _agg_body(y_hbm, meta_hbm, invc_hbm, za_hbm, out_hbm,
              mb0, mb1, mb2, mb3, rix0, rix1, rix2, rix3,
              wix0, wix1, wix2, wix3, div0, div1, div2, div3,
              wv0, wv1, wv2, wv3, rg0, rg1, rg2, rg3,
              m0, m1, m2, m3, g0, g1, g2, g3,
              s0, s1, s2, s3, w0, w1, w2, w3, acc_sh):
  cid = lax.axis_index("c")
  sid = lax.axis_index("s")
  wid = cid * NS + sid

  mbs = (mb0, mb1, mb2, mb3)
  msems = (m0, m1, m2, m3)
  rixs = (rix0, rix1, rix2, rix3)
  wixs = (wix0, wix1, wix2, wix3)
  divs = (div0, div1, div2, div3)
  wvs = (wv0, wv1, wv2, wv3)
  rgs = (rg0, rg1, rg2, rg3)
  gsems = (g0, g1, g2, g3)
  ssems = (s0, s1, s2, s3)
  wsems = (w0, w1, w2, w3)

  pltpu.sync_copy(za_hbm, rg0)
  for z in range(pl.cdiv(NZB, NS)):
    blk = sid + z * NS

    @pl.when(blk < NZB)
    def _():
      pltpu.sync_copy(rg0, acc_sh.at[pl.ds(blk * ZR, ZR)])

  def meta_issue(c, j):
    pltpu.async_copy(meta_hbm.at[wid, c], mbs[j], msems[j])

  def prep_issue(c, j):
    pltpu.make_async_copy(meta_hbm.at[wid, c], mbs[j], msems[j]).wait()
    for t in range(CH // 16):
      sl = pl.ds(t * 16, 16)
      rixs[j][sl] = mbs[j][2, sl] * N + mbs[j][0, sl]
      wixs[j][sl] = mbs[j][1, sl] * R + mbs[j][2, sl]
    pltpu.async_copy(invc_hbm.at[wixs[j]], wvs[j], wsems[j])
    pltpu.async_copy(y_hbm.at[rixs[j]], rgs[j], gsems[j])

  def process(c, q, first=False, last=False):
    qn = (q + 2) % 4
    pltpu.make_async_copy(y_hbm.at[rixs[q]], rgs[q], gsems[q]).wait()
    pltpu.make_async_copy(invc_hbm.at[wixs[q]], wvs[q], wsems[q]).wait()
    if not first:
      pltpu.make_async_copy(rgs[qn], acc_sh.at[divs[qn]], ssems[qn]).wait()
    if not last:
      @pl.when(c + 2 < NCHUNK)
      def _():
        prep_issue(c + 2, qn)

    for t in range(CH // 16):
      sl = pl.ds(t * 16, 16)
      divs[q][sl] = mbs[q][1, sl]
    if not last:
      @pl.when(c + 4 < NCHUNK)
      def _():
        meta_issue(c + 4, q)

    for t in range(CH // 16):
      sl = pl.ds(t * 16, 16)
      w16 = wvs[q][sl]
      for ii in range(16):
        w = w16[ii]
        for jj in range(D // 16):
          fl = pl.ds(jj * 16, 16)
          rgs[q][t * 16 + ii, fl] = rgs[q][t * 16 + ii, fl] * w
    pltpu.async_copy(rgs[q], acc_sh.at[divs[q]], ssems[q], add=True)

  for j in range(4):
    meta_issue(j, j)
  prep_issue(0, 0)
  prep_issue(1, 1)
  plsc.subcore_barrier()
  process(0, 0, first=True)
  process(1, 1, first=True)
  process(2, 2)
  process(3, 3)

  def quad(k, _):
    c = k * 4
    process(c, 0)
    process(c + 1, 1)
    process(c + 2, 2)
    process(c + 3, 3)
    return _

  lax.fori_loop(1, NCHUNK // 4, quad, None)
  process(NCHUNK - 1, 0, last=True)
  pltpu.make_async_copy(rgs[3], acc_sh.at[divs[3]], ssems[3]).wait()
  pltpu.make_async_copy(rgs[0], acc_sh.at[divs[0]], ssems[0]).wait()
  plsc.subcore_barrier()
  for z in range(pl.cdiv(NZB, NS)):
    blk = sid + z * NS

    @pl.when(blk < NZB)
    def _():
      pltpu.sync_copy(acc_sh.at[pl.ds(blk * ZR, ZR)], rg0)
      pltpu.sync_copy(rg0, out_hbm.at[cid, pl.ds(blk * ZR, ZR)])


def _aggregate(y, meta, invc, za):
  k = pl.kernel(
      _agg_body,
      name="agg_sc",
      out_type=jax.ShapeDtypeStruct((NC, N, D), jnp.float32),
      mesh=_mesh(),
      scratch_types=[
          pltpu.VMEM((3, CH), jnp.int32),
          pltpu.VMEM((3, CH), jnp.int32),
          pltpu.VMEM((3, CH), jnp.int32),
          pltpu.VMEM((3, CH), jnp.int32),
          pltpu.VMEM((CH,), jnp.int32),
          pltpu.VMEM((CH,), jnp.int32),
          pltpu.VMEM((CH,), jnp.int32),
          pltpu.VMEM((CH,), jnp.int32),
          pltpu.VMEM((CH,), jnp.int32),
          pltpu.VMEM((CH,), jnp.int32),
          pltpu.VMEM((CH,), jnp.int32),
          pltpu.VMEM((CH,), jnp.int32),
          pltpu.VMEM((CH,), jnp.int32),
          pltpu.VMEM((CH,), jnp.int32),
          pltpu.VMEM((CH,), jnp.int32),
          pltpu.VMEM((CH,), jnp.int32),
          pltpu.VMEM((CH,), jnp.float32),
          pltpu.VMEM((CH,), jnp.float32),
          pltpu.VMEM((CH,), jnp.float32),
          pltpu.VMEM((CH,), jnp.float32),
          pltpu.VMEM((CH, D), jnp.float32),
          pltpu.VMEM((CH, D), jnp.float32),
          pltpu.VMEM((CH, D), jnp.float32),
          pltpu.VMEM((CH, D), jnp.float32),
          pltpu.SemaphoreType.DMA,
          pltpu.SemaphoreType.DMA,
          pltpu.SemaphoreType.DMA,
          pltpu.SemaphoreType.DMA,
          pltpu.SemaphoreType.DMA,
          pltpu.SemaphoreType.DMA,
          pltpu.SemaphoreType.DMA,
          pltpu.SemaphoreType.DMA,
          pltpu.SemaphoreType.DMA,
          pltpu.SemaphoreType.DMA,
          pltpu.SemaphoreType.DMA,
          pltpu.SemaphoreType.DMA,
          pltpu.SemaphoreType.DMA,
          pltpu.SemaphoreType.DMA,
          pltpu.SemaphoreType.DMA,
          pltpu.SemaphoreType.DMA,
          pltpu.VMEM_SHARED((N, D), jnp.float32),
      ],
  )
  return k(y, meta, invc, za)


NB = 10
BN_ROWS = N // NB


def _inv_body(c_ref, o_ref):
  c = c_ref[0] + c_ref[1]
  o_ref[...] = 1.0 / jnp.maximum(c, 1.0)


def _inv_counts(counts):
  c2 = counts.reshape(NC, NR // 128, 128)
  out = pl.pallas_call(
      _inv_body,
      out_shape=jax.ShapeDtypeStruct((NR // 128, 128), jnp.float32),
  )(c2)
  return out.reshape(NR)


def _relmm_body(x_ref, w_ref, y_ref):
  y_ref[...] = jnp.dot(x_ref[...], w_ref[0],
                       preferred_element_type=jnp.float32)


def _rel_matmul(x, wrel):
  dout = wrel.shape[2]
  return pl.pallas_call(
      _relmm_body,
      grid=(R,),
      in_specs=[
          pl.BlockSpec((N, D), lambda r: (0, 0)),
          pl.BlockSpec((1, D, dout), lambda r: (r, 0, 0)),
      ],
      out_specs=pl.BlockSpec((N, dout), lambda r: (r, 0)),
      out_shape=jax.ShapeDtypeStruct((R * N, dout), jnp.float32),
  )(x, wrel)


def _pre_body(a_ref, x_ref, w_ref, b_ref, h_ref, p_ref):
  h = a_ref[0] + a_ref[1] + jnp.dot(x_ref[...], w_ref[...],
                                    preferred_element_type=jnp.float32)
  h = h + b_ref[...]
  h_ref[...] = h
  s = jnp.sum(h, axis=0, keepdims=True)
  ss = jnp.sum(h * h, axis=0, keepdims=True)
  p_ref[...] = jnp.concatenate(
      [s, ss, jnp.zeros((6, HID), jnp.float32)], axis=0)[None]


def _pre_bn(a, x, wroot, b):
  return pl.pallas_call(
      _pre_body,
      grid=(NB,),
      in_specs=[
          pl.BlockSpec((NC, BN_ROWS, HID), lambda i: (0, i, 0)),
          pl.BlockSpec((BN_ROWS, D), lambda i: (i, 0)),
          pl.BlockSpec((D, HID), lambda i: (0, 0)),
          pl.BlockSpec((1, HID), lambda i: (0, 0)),
      ],
      out_specs=[
          pl.BlockSpec((BN_ROWS, HID), lambda i: (i, 0)),
          pl.BlockSpec((1, 8, HID), lambda i: (i, 0, 0)),
      ],
      out_shape=[
          jax.ShapeDtypeStruct((N, HID), jnp.float32),
          jax.ShapeDtypeStruct((NB, 8, HID), jnp.float32),
      ],
  )(a, x, wroot, b)


def _bn_relu_body(h_ref, p_ref, g_ref, be_ref, o_ref):
  s = jnp.sum(p_ref[:, 0, :], axis=0)
  ss = jnp.sum(p_ref[:, 1, :], axis=0)
  m = s / N
  v = ss / N - m * m
  y = (h_ref[...] - m) * lax.rsqrt(v + EPS) * g_ref[...] + be_ref[...]
  o_ref[...] = jnp.maximum(y, 0.0)


def _bn_relu(h, parts, g, be):
  return pl.pallas_call(
      _bn_relu_body,
      grid=(NB,),
      in_specs=[
          pl.BlockSpec((BN_ROWS, HID), lambda i: (i, 0)),
          pl.BlockSpec((NB, 8, HID), lambda i: (0, 0, 0)),
          pl.BlockSpec((1, HID), lambda i: (0, 0)),
          pl.BlockSpec((1, HID), lambda i: (0, 0)),
      ],
      out_specs=pl.BlockSpec((BN_ROWS, HID), lambda i: (i, 0)),
      out_shape=jax.ShapeDtypeStruct((N, HID), jnp.float32),
  )(h, parts, g, be)


def _relmm_bn_body(h_ref, p_ref, g_ref, be_ref, w_ref, y_ref):
  su = jnp.sum(p_ref[:, 0, :], axis=0)
  ss = jnp.sum(p_ref[:, 1, :], axis=0)
  m = su / N
  v = ss / N - m * m
  hn = jnp.maximum(
      (h_ref[...] - m) * lax.rsqrt(v + EPS) * g_ref[...] + be_ref[...], 0.0)
  y_ref[...] = jnp.dot(hn, w_ref[0], preferred_element_type=jnp.float32)


def _rel_matmul_bn(h, parts, g, be, wrel):
  dout = wrel.shape[2]
  return pl.pallas_call(
      _relmm_bn_body,
      grid=(R,),
      in_specs=[
          pl.BlockSpec((N, HID), lambda r: (0, 0)),
          pl.BlockSpec((NB, 8, HID), lambda r: (0, 0, 0)),
          pl.BlockSpec((1, HID), lambda r: (0, 0)),
          pl.BlockSpec((1, HID), lambda r: (0, 0)),
          pl.BlockSpec((1, HID, dout), lambda r: (r, 0, 0)),
      ],
      out_specs=pl.BlockSpec((N, dout), lambda r: (r, 0)),
      out_shape=jax.ShapeDtypeStruct((R * N, dout), jnp.float32),
  )(h, parts, g, be, wrel)


def _pre_bn2_body(a_ref, h_ref, p_ref, g_ref, be_ref, w_ref, b_ref,
                  h2_ref, p2_ref):
  su = jnp.sum(p_ref[:, 0, :], axis=0)
  ss = jnp.sum(p_ref[:, 1, :], axis=0)
  m = su / N
  v = ss / N - m * m
  hn = jnp.maximum(
      (h_ref[...] - m) * lax.rsqrt(v + EPS) * g_ref[...] + be_ref[...], 0.0)
  h = a_ref[0] + a_ref[1] + jnp.dot(hn, w_ref[...],
                                    preferred_element_type=jnp.float32)
  h = h + b_ref[...]
  h2_ref[...] = h
  sb = jnp.sum(h, axis=0, keepdims=True)
  sq = jnp.sum(h * h, axis=0, keepdims=True)
  p2_ref[...] = jnp.concatenate(
      [sb, sq, jnp.zeros((6, HID), jnp.float32)], axis=0)[None]


def _pre_bn2(a, h, parts, g, be, wroot, b):
  return pl.pallas_call(
      _pre_bn2_body,
      grid=(NB,),
      in_specs=[
          pl.BlockSpec((NC, BN_ROWS, HID), lambda i: (0, i, 0)),
          pl.BlockSpec((BN_ROWS, HID), lambda i: (i, 0)),
          pl.BlockSpec((NB, 8, HID), lambda i: (0, 0, 0)),
          pl.BlockSpec((1, HID), lambda i: (0, 0)),
          pl.BlockSpec((1, HID), lambda i: (0, 0)),
          pl.BlockSpec((HID, HID), lambda i: (0, 0)),
          pl.BlockSpec((1, HID), lambda i: (0, 0)),
      ],
      out_specs=[
          pl.BlockSpec((BN_ROWS, HID), lambda i: (i, 0)),
          pl.BlockSpec((1, 8, HID), lambda i: (i, 0, 0)),
      ],
      out_shape=[
          jax.ShapeDtypeStruct((N, HID), jnp.float32),
          jax.ShapeDtypeStruct((NB, 8, HID), jnp.float32),
      ],
  )(a, h, parts, g, be, wroot, b)


def _bn_relu_fc_body(h_ref, p_ref, g_ref, be_ref, wf_ref, bf_ref, o_ref):
  s = jnp.sum(p_ref[:, 0, :], axis=0)
  ss = jnp.sum(p_ref[:, 1, :], axis=0)
  m = s / N
  v = ss / N - m * m
  y = (h_ref[...] - m) * lax.rsqrt(v + EPS) * g_ref[...] + be_ref[...]
  y = jnp.maximum(y, 0.0)
  o_ref[...] = jnp.dot(y, wf_ref[...],
                       preferred_element_type=jnp.float32) + bf_ref[...]


def _bn_relu_fc(h, parts, g, be, wf, bf):
  return pl.pallas_call(
      _bn_relu_fc_body,
      grid=(NB,),
      in_specs=[
          pl.BlockSpec((BN_ROWS, HID), lambda i: (i, 0)),
          pl.BlockSpec((NB, 8, HID), lambda i: (0, 0, 0)),
          pl.BlockSpec((1, HID), lambda i: (0, 0)),
          pl.BlockSpec((1, HID), lambda i: (0, 0)),
          pl.BlockSpec((HID, NUM_CLASSES), lambda i: (0, 0)),
          pl.BlockSpec((1, NUM_CLASSES), lambda i: (0, 0)),
      ],
      out_specs=pl.BlockSpec((BN_ROWS, NUM_CLASSES), lambda i: (i, 0)),
      out_shape=jax.ShapeDtypeStruct((N, NUM_CLASSES), jnp.float32),
  )(h, parts, g, be, wf, bf)


def kernel(x, edge_index, edge_attr, Wrel1, Wroot1, b1, g1, be1,
           Wrel2, Wroot2, b2, g2, be2, Wf, bf):
  src = edge_index[0].astype(jnp.int32)
  dst = edge_index[1].astype(jnp.int32)
  et = edge_attr.astype(jnp.int32)
  zc = jnp.zeros((NR // NS,), jnp.float32)
  za = jnp.zeros((ZR, D), jnp.float32)

  counts, meta = _counts(src, dst, et, zc)
  meta = meta.reshape(NW, NCHUNK, 3, CH)
  invc = _inv_counts(counts)

  b1r = b1.reshape(1, HID)
  g1r = g1.reshape(1, HID)
  be1r = be1.reshape(1, HID)
  b2r = b2.reshape(1, HID)
  g2r = g2.reshape(1, HID)
  be2r = be2.reshape(1, HID)
  bfr = bf.reshape(1, NUM_CLASSES)

  y1 = _rel_matmul(x, Wrel1)
  a1 = _aggregate(y1, meta, invc, za)
  h1p, p1 = _pre_bn(a1, x, Wroot1, b1r)

  y2 = _rel_matmul_bn(h1p, p1, g1r, be1r, Wrel2)
  a2 = _aggregate(y2, meta, invc, za)
  h2p, p2 = _pre_bn2(a2, h1p, p1, g1r, be1r, Wroot2, b2r)
  out = _bn_relu_fc(h2p, p2, g2r, be2r, Wf, bfr)
  return out
